# baseline (speedup 1.0000x reference)
import functools

import numpy as _np

import jax
import jax.numpy as jnp
from jax import lax
from jax.experimental import pallas as pl
from jax.experimental.pallas import tpu as pltpu
from jax.experimental.pallas import tpu_sc as plsc

N_NODES = 10000
N_EDGES = 320000
NPAD = 10016
ROWS_PER_TILE = NPAD // 16
PAD_DST = 10008
N_WORKERS = 32
EPW = N_EDGES // N_WORKERS
CHUNK = 128
CPW = 80
ZSIZES = (128, 128, 128, 128, 114)


def _make_seg_sum(d, with_deg, col_split, nb):
  cpw = 2 * CPW if col_split else CPW
  ng = cpw // nb
  mesh = plsc.VectorSubcoreMesh(core_axis_name="c", subcore_axis_name="s")
  out_type = [jax.ShapeDtypeStruct((2, NPAD, d), jnp.float32)]
  scratch = [
      pltpu.VMEM((cpw, CHUNK), jnp.int32),
      pltpu.VMEM((cpw, CHUNK), jnp.int32),
      pltpu.VMEM((nb, CHUNK, d), jnp.bfloat16),
      pltpu.VMEM((nb, CHUNK, d), jnp.float32),
      pltpu.VMEM_SHARED((NPAD, d), jnp.float32),
      pltpu.SemaphoreType.DMA((nb,)),
      pltpu.SemaphoreType.DMA((nb,)),
  ]
  if with_deg:
    out_type.append(jax.ShapeDtypeStruct((2, NPAD, 16), jnp.float32))
    scratch += [
        pltpu.VMEM((CHUNK, 16), jnp.float32),
        pltpu.VMEM_SHARED((NPAD, 16), jnp.float32),
    ]

  def body(table, srcp, dstp, *rest):
    if with_deg:
      out, dego, sidx, didx, rows16, rows, acc, gsem, ssem, ones, dacc = rest
    else:
      out, sidx, didx, rows16, rows, acc, gsem, ssem = rest
    cc = lax.axis_index("c")
    ss = lax.axis_index("s")
    zvec = jnp.zeros((16,), jnp.float32)
    rbase = pl.multiple_of(ss * ROWS_PER_TILE, 2)

    widx = ss if col_split else cc * 16 + ss
    pltpu.async_copy(srcp.at[widx], sidx, gsem.at[0])
    pltpu.async_copy(dstp.at[widx], didx, gsem.at[1])

    def zrows(i, carry):
      for k in range(d // 16):
        rows[0, i, pl.ds(k * 16, 16)] = zvec
      return carry
    lax.fori_loop(0, CHUNK, zrows, 0)
    off = 0
    for sz in ZSIZES:
      pltpu.sync_copy(rows.at[0, pl.ds(0, sz)],
                      acc.at[pl.ds(rbase + off, sz)])
      off += sz
    if with_deg:
      def z16(i, carry):
        ones[i] = zvec
        return carry
      lax.fori_loop(0, CHUNK, z16, 0)
      off = 0
      for sz in ZSIZES:
        pltpu.sync_copy(ones.at[pl.ds(0, sz)],
                        dacc.at[pl.ds(rbase + off, sz)])
        off += sz
      def o16(i, carry):
        ones[i] = jnp.ones((16,), jnp.float32)
        return carry
      lax.fori_loop(0, CHUNK, o16, 0)

    pltpu.make_async_copy(srcp.at[widx], sidx, gsem.at[0]).wait()
    pltpu.make_async_copy(dstp.at[widx], didx, gsem.at[1]).wait()

    tbl = table.at[cc]

    def gather_start(b, j):
      pltpu.async_copy(tbl.at[sidx.at[j]], rows16.at[b], gsem.at[b])

    def gather_wait(b):
      pltpu.make_async_copy(tbl.at[sidx.at[0]], rows16.at[b],
                            gsem.at[b]).wait()

    def convert(b):
      def cbody(i, carry):
        for u in range(4):
          for k in range(d // 32):
            ab = rows16[b, i * 4 + u, pl.ds(32 * k, 32)]
            lo, hi = plsc.unpack(ab, format=plsc.PackFormat.INTERLEAVED)
            rows[b, i * 4 + u, pl.ds(32 * k, 16)] = lo
            rows[b, i * 4 + u, pl.ds(32 * k + 16, 16)] = hi
        return carry
      lax.fori_loop(0, CHUNK // 4, cbody, 0)

    def scatter_start(b, j):
      pltpu.async_copy(rows.at[b], acc.at[didx.at[j]], ssem.at[b],
                       add=True)
      if with_deg:
        pltpu.async_copy(ones, dacc.at[didx.at[j]], ssem.at[b], add=True)

    def scatter_wait(b):
      pltpu.make_async_copy(rows.at[b], acc.at[didx.at[0]],
                            ssem.at[b]).wait()
      if with_deg:
        pltpu.make_async_copy(ones, dacc.at[didx.at[0]],
                              ssem.at[b]).wait()

    plsc.subcore_barrier()

    for b in range(nb):
      gather_start(b, b)

    def group(g, carry):
      base = g * nb
      for b in range(nb):
        gather_wait(b)
        convert(b)
        scatter_start(b, base + b)
      for b in range(nb):
        @pl.when(g < ng - 1)
        def _():
          scatter_wait(b)
          gather_start(b, base + nb + b)
      return carry
    lax.fori_loop(0, ng, group, 0)
    for b in range(nb):
      scatter_wait(b)

    plsc.subcore_barrier()
    pltpu.sync_copy(acc.at[pl.ds(rbase, ROWS_PER_TILE)],
                    out.at[cc, pl.ds(rbase, ROWS_PER_TILE)])
    if with_deg:
      pltpu.sync_copy(dacc.at[pl.ds(rbase, ROWS_PER_TILE)],
                      dego.at[cc, pl.ds(rbase, ROWS_PER_TILE)])

  return pl.kernel(
      body, mesh=mesh, out_type=out_type, scratch_types=scratch,
      compiler_params=pltpu.CompilerParams(use_tc_tiling_on_sc=False,
                                           needs_layout_passes=False))


_BLK = 1000
_GRID = N_NODES // _BLK


def _mm_body(x_ref, w_ref, o_ref):
  r = jnp.dot(x_ref[...], w_ref[...], preferred_element_type=jnp.float32)
  r = r.astype(jnp.bfloat16)
  o_ref[0] = r
  o_ref[1] = r


def _project(x, w):
  din, dout = w.shape
  return pl.pallas_call(
      _mm_body,
      grid=(_GRID,),
      in_specs=[
          pl.BlockSpec((_BLK, din), lambda i: (i, 0)),
          pl.BlockSpec((din, dout), lambda i: (0, 0)),
      ],
      out_specs=pl.BlockSpec((2, _BLK, dout), lambda i: (0, i, 0)),
      out_shape=jax.ShapeDtypeStruct((2, N_NODES, dout), jnp.bfloat16),
  )(x, w)


def _rdeg(dacc_blk):
  deg = dacc_blk[0, :, 0:1] + dacc_blk[1, :, 0:1]
  return 1.0 / jnp.clip(deg, 1.0, None)


def _comb1_body(agg_ref, dacc_ref, x_ref, w_ref, b_ref, oh_ref, of_ref):
  a = agg_ref[...]
  mean = (a[0] + a[1]) * _rdeg(dacc_ref[...])
  o = mean + jnp.dot(x_ref[...], w_ref[...],
                     preferred_element_type=jnp.float32) + b_ref[...]
  r = jnp.maximum(o, 0.0)
  h = r.shape[-1] // 2
  rh = r.astype(jnp.bfloat16)
  oh_ref[0] = rh[:, :h]
  oh_ref[1] = rh[:, h:]
  of_ref[...] = r


def _comb_body(relu, concat, halves, agg_ref, dacc_ref, h_ref, wl_ref,
               wr_ref, b_ref, *o_refs):
  a = agg_ref[...]
  if concat:
    s = jnp.concatenate([a[0], a[1]], axis=1)
  else:
    s = a[0] + a[1]
  mean = s * _rdeg(dacc_ref[...])
  o = (jnp.dot(mean, wl_ref[...], preferred_element_type=jnp.float32)
       + jnp.dot(h_ref[...], wr_ref[...], preferred_element_type=jnp.float32)
       + b_ref[...])
  r = jnp.maximum(o, 0.0) if relu else o
  if halves:
    hw = r.shape[-1] // 2
    rh = r.astype(jnp.bfloat16)
    o_refs[0][0] = rh[:, :hw]
    o_refs[0][1] = rh[:, hw:]
    o_refs[1][...] = r
  else:
    o_refs[0][...] = r


def _combine1(agg, dacc, x, w, b):
  dout = w.shape[1]
  return pl.pallas_call(
      _comb1_body,
      grid=(_GRID,),
      in_specs=[
          pl.BlockSpec((2, _BLK, dout), lambda i: (0, i, 0)),
          pl.BlockSpec((2, _BLK, 16), lambda i: (0, i, 0)),
          pl.BlockSpec((_BLK, w.shape[0]), lambda i: (i, 0)),
          pl.BlockSpec(w.shape, lambda i: (0, 0)),
          pl.BlockSpec((1, dout), lambda i: (0, 0)),
      ],
      out_specs=[
          pl.BlockSpec((2, _BLK, dout // 2), lambda i: (0, i, 0)),
          pl.BlockSpec((_BLK, dout), lambda i: (i, 0)),
      ],
      out_shape=[
          jax.ShapeDtypeStruct((2, N_NODES, dout // 2), jnp.bfloat16),
          jax.ShapeDtypeStruct((N_NODES, dout), jnp.float32),
      ],
  )(agg, dacc, x, w, b.reshape(1, dout))


def _combine(agg, dacc, h, wl, wr, b, relu, concat=False, halves=False):
  din, dout = wl.shape
  dh = h.shape[1]
  da = din // 2 if concat else din
  if halves:
    out_specs = [
        pl.BlockSpec((2, _BLK, dout // 2), lambda i: (0, i, 0)),
        pl.BlockSpec((_BLK, dout), lambda i: (i, 0)),
    ]
    out_shape = [
        jax.ShapeDtypeStruct((2, N_NODES, dout // 2), jnp.bfloat16),
        jax.ShapeDtypeStruct((N_NODES, dout), jnp.float32),
    ]
  else:
    out_specs = pl.BlockSpec((_BLK, dout), lambda i: (i, 0))
    out_shape = jax.ShapeDtypeStruct((N_NODES, dout), jnp.float32)
  return pl.pallas_call(
      functools.partial(_comb_body, relu, concat, halves),
      grid=(_GRID,),
      in_specs=[
          pl.BlockSpec((2, _BLK, da), lambda i: (0, i, 0)),
          pl.BlockSpec((2, _BLK, 16), lambda i: (0, i, 0)),
          pl.BlockSpec((_BLK, dh), lambda i: (i, 0)),
          pl.BlockSpec((din, dout), lambda i: (0, 0)),
          pl.BlockSpec((dh, dout), lambda i: (0, 0)),
          pl.BlockSpec((1, dout), lambda i: (0, 0)),
      ],
      out_specs=out_specs,
      out_shape=out_shape,
  )(agg, dacc, h, wl, wr, b.reshape(1, dout))


_seg64_deg = _make_seg_sum(64, with_deg=True, col_split=False, nb=4)
_seg32c = _make_seg_sum(32, with_deg=False, col_split=True, nb=8)
_seg64c = _make_seg_sum(64, with_deg=False, col_split=True, nb=4)


def _qperm(n):
  q = []
  for g in range(0, n, 32):
    q += [g + 2 * t for t in range(16)]
    q += [g + 2 * t + 1 for t in range(16)]
  return _np.array(q)


_Q32 = _qperm(32)
_Q64 = _qperm(64)
_Q64INV = _np.argsort(_Q64)
_PERM2 = _np.concatenate([_Q32, 32 + _Q32])
_PERM3 = _np.concatenate([_Q64, 64 + _Q64])


def kernel(x, edge_index, W1l, W1r, b1, W2l, W2r, b2, W3l, W3r, b3):
  src = edge_index[0].astype(jnp.int32).reshape(N_WORKERS, EPW)
  dst = edge_index[1].astype(jnp.int32).reshape(N_WORKERS, EPW)
  npad = CPW * CHUNK - EPW
  srcp = jnp.concatenate(
      [src, jnp.zeros((N_WORKERS, npad), jnp.int32)], axis=1
  ).reshape(N_WORKERS, CPW, CHUNK)
  dstp = jnp.concatenate(
      [dst, jnp.full((N_WORKERS, npad), PAD_DST, jnp.int32)], axis=1
  ).reshape(N_WORKERS, CPW, CHUNK)
  srcp2 = srcp.reshape(16, 2 * CPW, CHUNK)
  dstp2 = dstp.reshape(16, 2 * CPW, CHUNK)

  p1 = _project(x, W1l[:, _Q64INV])
  agg1, dacc = _seg64_deg(p1, srcp, dstp)
  h1h, h1 = _combine1(agg1, dacc, x, W1r, b1)

  agg2, = _seg32c(h1h, srcp2, dstp2)
  h2h, h2 = _combine(agg2, dacc, h1, W2l[_PERM2, :], W2r, b2, relu=True,
                     concat=True, halves=True)

  agg3, = _seg64c(h2h, srcp2, dstp2)
  out = _combine(agg3, dacc, h2, W3l[_PERM3, :], W3r, b3, relu=False,
                 concat=True)
  return out


# speedup vs baseline: 1.1777x; 1.1777x over previous
import functools

import jax
import jax.numpy as jnp
from jax import lax
from jax.experimental import pallas as pl
from jax.experimental.pallas import tpu as pltpu
from jax.experimental.pallas import tpu_sc as plsc

N_NODES = 10000
N_EDGES = 320000
NPAD = 10016
ROWS_PER_TILE = NPAD // 16
PAD_DST = 10008
N_WORKERS = 32
EPW = N_EDGES // N_WORKERS
CHUNK = 128
CPW = 80
ZSIZES = (128, 128, 128, 128, 114)


def _make_seg_sum(d, with_deg, col_split, nb):
  cpw = 2 * CPW if col_split else CPW
  ng = cpw // nb
  mesh = plsc.VectorSubcoreMesh(core_axis_name="c", subcore_axis_name="s")
  out_type = [jax.ShapeDtypeStruct((2, NPAD, d), jnp.float32)]
  scratch = [
      pltpu.VMEM((cpw, CHUNK), jnp.int32),
      pltpu.VMEM((cpw, CHUNK), jnp.int32),
      pltpu.VMEM((nb, CHUNK, d), jnp.float32),
      pltpu.VMEM_SHARED((NPAD, d), jnp.float32),
      pltpu.SemaphoreType.DMA((nb,)),
      pltpu.SemaphoreType.DMA((nb,)),
  ]
  if with_deg:
    out_type.append(jax.ShapeDtypeStruct((2, NPAD, 16), jnp.float32))
    scratch += [
        pltpu.VMEM((CHUNK, 16), jnp.float32),
        pltpu.VMEM_SHARED((NPAD, 16), jnp.float32),
    ]

  def body(table, srcp, dstp, *rest):
    if with_deg:
      out, dego, sidx, didx, rows, acc, gsem, ssem, ones, dacc = rest
    else:
      out, sidx, didx, rows, acc, gsem, ssem = rest
    cc = lax.axis_index("c")
    ss = lax.axis_index("s")
    zvec = jnp.zeros((16,), jnp.float32)
    rbase = pl.multiple_of(ss * ROWS_PER_TILE, 2)

    widx = ss if col_split else cc * 16 + ss
    pltpu.async_copy(srcp.at[widx], sidx, gsem.at[0])
    pltpu.async_copy(dstp.at[widx], didx, gsem.at[1])

    def zrows(i, carry):
      for k in range(d // 16):
        rows[0, i, pl.ds(k * 16, 16)] = zvec
      return carry
    lax.fori_loop(0, CHUNK, zrows, 0)
    off = 0
    for sz in ZSIZES:
      pltpu.sync_copy(rows.at[0, pl.ds(0, sz)],
                      acc.at[pl.ds(rbase + off, sz)])
      off += sz
    if with_deg:
      def z16(i, carry):
        ones[i] = zvec
        return carry
      lax.fori_loop(0, CHUNK, z16, 0)
      off = 0
      for sz in ZSIZES:
        pltpu.sync_copy(ones.at[pl.ds(0, sz)],
                        dacc.at[pl.ds(rbase + off, sz)])
        off += sz
      def o16(i, carry):
        ones[i] = jnp.ones((16,), jnp.float32)
        return carry
      lax.fori_loop(0, CHUNK, o16, 0)

    pltpu.make_async_copy(srcp.at[widx], sidx, gsem.at[0]).wait()
    pltpu.make_async_copy(dstp.at[widx], didx, gsem.at[1]).wait()

    tbl = table.at[cc]

    def gather_start(b, j):
      pltpu.async_copy(tbl.at[sidx.at[j]], rows.at[b], gsem.at[b])

    def gather_wait(b):
      pltpu.make_async_copy(tbl.at[sidx.at[0]], rows.at[b],
                            gsem.at[b]).wait()

    def scatter_start(b, j):
      pltpu.async_copy(rows.at[b], acc.at[didx.at[j]], ssem.at[b],
                       add=True)
      if with_deg:
        pltpu.async_copy(ones, dacc.at[didx.at[j]], ssem.at[b], add=True)

    def scatter_wait(b):
      pltpu.make_async_copy(rows.at[b], acc.at[didx.at[0]],
                            ssem.at[b]).wait()
      if with_deg:
        pltpu.make_async_copy(ones, dacc.at[didx.at[0]],
                              ssem.at[b]).wait()

    plsc.subcore_barrier()

    for b in range(nb):
      gather_start(b, b)

    def group(g, carry):
      base = g * nb
      for b in range(nb):
        gather_wait(b)
        scatter_start(b, base + b)
      for b in range(nb):
        @pl.when(g < ng - 1)
        def _():
          scatter_wait(b)
          gather_start(b, base + nb + b)
      return carry
    lax.fori_loop(0, ng, group, 0)
    for b in range(nb):
      scatter_wait(b)

    plsc.subcore_barrier()
    pltpu.sync_copy(acc.at[pl.ds(rbase, ROWS_PER_TILE)],
                    out.at[cc, pl.ds(rbase, ROWS_PER_TILE)])
    if with_deg:
      pltpu.sync_copy(dacc.at[pl.ds(rbase, ROWS_PER_TILE)],
                      dego.at[cc, pl.ds(rbase, ROWS_PER_TILE)])

  return pl.kernel(
      body, mesh=mesh, out_type=out_type, scratch_types=scratch,
      compiler_params=pltpu.CompilerParams(use_tc_tiling_on_sc=False,
                                           needs_layout_passes=False))


_BLK = 1000
_GRID = N_NODES // _BLK


def _mm_body(x_ref, w_ref, o_ref):
  r = jnp.dot(x_ref[...], w_ref[...], preferred_element_type=jnp.float32)
  o_ref[0] = r
  o_ref[1] = r


def _project(x, w):
  din, dout = w.shape
  return pl.pallas_call(
      _mm_body,
      grid=(_GRID,),
      in_specs=[
          pl.BlockSpec((_BLK, din), lambda i: (i, 0)),
          pl.BlockSpec((din, dout), lambda i: (0, 0)),
      ],
      out_specs=pl.BlockSpec((2, _BLK, dout), lambda i: (0, i, 0)),
      out_shape=jax.ShapeDtypeStruct((2, N_NODES, dout), jnp.float32),
  )(x, w)


def _rdeg(dacc_blk):
  deg = dacc_blk[0, :, 0:1] + dacc_blk[1, :, 0:1]
  return 1.0 / jnp.clip(deg, 1.0, None)


def _comb1_body(agg_ref, dacc_ref, x_ref, w_ref, b_ref, oh_ref, of_ref):
  a = agg_ref[...]
  mean = (a[0] + a[1]) * _rdeg(dacc_ref[...])
  o = mean + jnp.dot(x_ref[...], w_ref[...],
                     preferred_element_type=jnp.float32) + b_ref[...]
  r = jnp.maximum(o, 0.0)
  h = r.shape[-1] // 2
  oh_ref[0] = r[:, :h]
  oh_ref[1] = r[:, h:]
  of_ref[...] = r


def _comb_body(relu, concat, halves, agg_ref, dacc_ref, h_ref, wl_ref,
               wr_ref, b_ref, *o_refs):
  a = agg_ref[...]
  if concat:
    s = jnp.concatenate([a[0], a[1]], axis=1)
  else:
    s = a[0] + a[1]
  mean = s * _rdeg(dacc_ref[...])
  o = (jnp.dot(mean, wl_ref[...], preferred_element_type=jnp.float32)
       + jnp.dot(h_ref[...], wr_ref[...], preferred_element_type=jnp.float32)
       + b_ref[...])
  r = jnp.maximum(o, 0.0) if relu else o
  if halves:
    hw = r.shape[-1] // 2
    o_refs[0][0] = r[:, :hw]
    o_refs[0][1] = r[:, hw:]
    o_refs[1][...] = r
  else:
    o_refs[0][...] = r


def _combine1(agg, dacc, x, w, b):
  dout = w.shape[1]
  return pl.pallas_call(
      _comb1_body,
      grid=(_GRID,),
      in_specs=[
          pl.BlockSpec((2, _BLK, dout), lambda i: (0, i, 0)),
          pl.BlockSpec((2, _BLK, 16), lambda i: (0, i, 0)),
          pl.BlockSpec((_BLK, w.shape[0]), lambda i: (i, 0)),
          pl.BlockSpec(w.shape, lambda i: (0, 0)),
          pl.BlockSpec((1, dout), lambda i: (0, 0)),
      ],
      out_specs=[
          pl.BlockSpec((2, _BLK, dout // 2), lambda i: (0, i, 0)),
          pl.BlockSpec((_BLK, dout), lambda i: (i, 0)),
      ],
      out_shape=[
          jax.ShapeDtypeStruct((2, N_NODES, dout // 2), jnp.float32),
          jax.ShapeDtypeStruct((N_NODES, dout), jnp.float32),
      ],
  )(agg, dacc, x, w, b.reshape(1, dout))


def _combine(agg, dacc, h, wl, wr, b, relu, concat=False, halves=False):
  din, dout = wl.shape
  dh = h.shape[1]
  da = din // 2 if concat else din
  if halves:
    out_specs = [
        pl.BlockSpec((2, _BLK, dout // 2), lambda i: (0, i, 0)),
        pl.BlockSpec((_BLK, dout), lambda i: (i, 0)),
    ]
    out_shape = [
        jax.ShapeDtypeStruct((2, N_NODES, dout // 2), jnp.float32),
        jax.ShapeDtypeStruct((N_NODES, dout), jnp.float32),
    ]
  else:
    out_specs = pl.BlockSpec((_BLK, dout), lambda i: (i, 0))
    out_shape = jax.ShapeDtypeStruct((N_NODES, dout), jnp.float32)
  return pl.pallas_call(
      functools.partial(_comb_body, relu, concat, halves),
      grid=(_GRID,),
      in_specs=[
          pl.BlockSpec((2, _BLK, da), lambda i: (0, i, 0)),
          pl.BlockSpec((2, _BLK, 16), lambda i: (0, i, 0)),
          pl.BlockSpec((_BLK, dh), lambda i: (i, 0)),
          pl.BlockSpec((din, dout), lambda i: (0, 0)),
          pl.BlockSpec((dh, dout), lambda i: (0, 0)),
          pl.BlockSpec((1, dout), lambda i: (0, 0)),
      ],
      out_specs=out_specs,
      out_shape=out_shape,
  )(agg, dacc, h, wl, wr, b.reshape(1, dout))


_seg64_deg = _make_seg_sum(64, with_deg=True, col_split=False, nb=5)
_seg32c = _make_seg_sum(32, with_deg=False, col_split=True, nb=8)
_seg64c = _make_seg_sum(64, with_deg=False, col_split=True, nb=5)


def kernel(x, edge_index, W1l, W1r, b1, W2l, W2r, b2, W3l, W3r, b3):
  src = edge_index[0].astype(jnp.int32).reshape(N_WORKERS, EPW)
  dst = edge_index[1].astype(jnp.int32).reshape(N_WORKERS, EPW)
  npad = CPW * CHUNK - EPW
  srcp = jnp.concatenate(
      [src, jnp.zeros((N_WORKERS, npad), jnp.int32)], axis=1
  ).reshape(N_WORKERS, CPW, CHUNK)
  dstp = jnp.concatenate(
      [dst, jnp.full((N_WORKERS, npad), PAD_DST, jnp.int32)], axis=1
  ).reshape(N_WORKERS, CPW, CHUNK)
  srcp2 = srcp.reshape(16, 2 * CPW, CHUNK)
  dstp2 = dstp.reshape(16, 2 * CPW, CHUNK)

  p1 = _project(x, W1l)
  agg1, dacc = _seg64_deg(p1, srcp, dstp)
  h1h, h1 = _combine1(agg1, dacc, x, W1r, b1)

  agg2, = _seg32c(h1h, srcp2, dstp2)
  h2h, h2 = _combine(agg2, dacc, h1, W2l, W2r, b2, relu=True,
                     concat=True, halves=True)

  agg3, = _seg64c(h2h, srcp2, dstp2)
  out = _combine(agg3, dacc, h2, W3l, W3r, b3, relu=False, concat=True)
  return out
